# Initial kernel scaffold; baseline (speedup 1.0000x reference)
import functools

import jax
import jax.numpy as jnp
from jax import lax
from jax.experimental import pallas as pl
from jax.experimental.pallas import tpu as pltpu
from jax.experimental.pallas import tpu_sc as plsc

N = 10000
E = 320000
D = 128
CELL_LEN = 100
SAME_W = 0.3
CROSS_W = 1.0

NC = 2
NS = 16
NW = NC * NS
EPW = E // NW
C = 80
NCHUNK = EPW // C
N_PAD = 10240
ZPT = N_PAD // NS
LANES = 16


def _sc_agg(with_counts):
  mesh = plsc.VectorSubcoreMesh(core_axis_name="c", subcore_axis_name="s")
  out_type = [jax.ShapeDtypeStruct((NC, N_PAD, D), jnp.float32)]
  if with_counts:
    out_type.append(jax.ShapeDtypeStruct((NC, N_PAD), jnp.float32))
  scratch = [
      pltpu.VMEM_SHARED((N_PAD, D), jnp.float32),
      pltpu.VMEM_SHARED((N_PAD,), jnp.float32),
      pltpu.VMEM((C,), jnp.int32),
      pltpu.VMEM((C,), jnp.int32),
      pltpu.VMEM((C,), jnp.float32),
      pltpu.VMEM((C,), jnp.float32),
      pltpu.VMEM((C,), jnp.float32),
      pltpu.VMEM((C, D), jnp.float32),
  ]

  def body(row_h, col_h, ew_h, x_h, zrow_h, zcnt_h, *rest):
    if with_counts:
      out_acc, out_cnt = rest[0], rest[1]
      scr = rest[2:]
    else:
      out_acc = rest[0]
      scr = rest[1:]
    acc_s, cnt_s, idx_r, idx_c, ew_v, w_v, ones_v, rows_v = scr

    cid = lax.axis_index("c")
    sid = lax.axis_index("s")
    wid = sid * NC + cid

    lo = sid * ZPT
    pltpu.sync_copy(zrow_h.at[pl.ds(lo, ZPT), :], acc_s.at[pl.ds(lo, ZPT), :])
    if with_counts:
      pltpu.sync_copy(zcnt_h.at[pl.ds(lo, ZPT)], cnt_s.at[pl.ds(lo, ZPT)])
    for j in range(C // LANES):
      ones_v[pl.ds(j * LANES, LANES)] = jnp.ones((LANES,), jnp.float32)
    plsc.subcore_barrier()

    def chunk_body(k, carry):
      base = pl.multiple_of(wid * EPW + k * C, 8)
      pltpu.sync_copy(row_h.at[pl.ds(base, C)], idx_r)
      pltpu.sync_copy(col_h.at[pl.ds(base, C)], idx_c)
      pltpu.sync_copy(ew_h.at[pl.ds(base, C)], ew_v)

      for j in range(C // LANES):
        sl = pl.ds(j * LANES, LANES)
        r16 = idx_r[sl]
        c16 = idx_c[sl]
        same = (r16 <= CELL_LEN) == (c16 <= CELL_LEN)
        tw = jnp.where(same, jnp.float32(SAME_W), jnp.float32(CROSS_W))
        w_v[sl] = tw * ew_v[sl]

      pltpu.sync_copy(x_h.at[idx_r], rows_v)

      def scale_row(r, c2):
        widx = jnp.zeros((LANES,), jnp.int32) + r
        wr = plsc.load_gather(w_v, [widx])
        for d in range(D // LANES):
          sl = pl.ds(d * LANES, LANES)
          rows_v[r, sl] = rows_v[r, sl] * wr
        return c2

      lax.fori_loop(0, C, scale_row, 0)

      pltpu.sync_copy(rows_v, acc_s.at[idx_c], add=True)
      if with_counts:
        pltpu.sync_copy(ones_v, cnt_s.at[idx_c], add=True)
      return carry

    lax.fori_loop(0, NCHUNK, chunk_body, 0)
    plsc.subcore_barrier()

    pltpu.sync_copy(acc_s.at[pl.ds(lo, ZPT), :], out_acc.at[cid, pl.ds(lo, ZPT), :])
    if with_counts:
      pltpu.sync_copy(cnt_s.at[pl.ds(lo, ZPT)], out_cnt.at[cid, pl.ds(lo, ZPT)])

  return pl.kernel(body, out_type=out_type, mesh=mesh, scratch_types=scratch)


_sc_agg_counts = _sc_agg(True)
_sc_agg_plain = _sc_agg(False)


BN = 640


def _combine_body(acc_ref, cnt_ref, x_ref, wt_ref, o_ref):
  acc = acc_ref[0] + acc_ref[1]
  cnt = cnt_ref[0] + cnt_ref[1]
  inv = 1.0 / jnp.maximum(cnt, 1.0)
  o_ref[...] = acc * inv + jnp.dot(
      x_ref[...], wt_ref[...], preferred_element_type=jnp.float32)


def _tc_combine(acc_p, cnt_p, x_pad, wt):
  grid = (N_PAD // BN,)
  return pl.pallas_call(
      _combine_body,
      grid=grid,
      in_specs=[
          pl.BlockSpec((NC, BN, D), lambda i: (0, i, 0)),
          pl.BlockSpec((NC, BN, 1), lambda i: (0, i, 0)),
          pl.BlockSpec((BN, D), lambda i: (i, 0)),
          pl.BlockSpec((D, D), lambda i: (0, 0)),
      ],
      out_specs=pl.BlockSpec((BN, D), lambda i: (i, 0)),
      out_shape=jax.ShapeDtypeStruct((N_PAD, D), jnp.float32),
  )(acc_p, cnt_p, x_pad, wt)


def kernel(x, edge_index, edge_weight, W1, W2):
  row = edge_index[0]
  col = edge_index[1]
  x_pad = jnp.pad(x, ((0, N_PAD - N), (0, 0)))
  zrow = jnp.zeros((N_PAD, D), jnp.float32)
  zcnt = jnp.zeros((N_PAD,), jnp.float32)

  acc_p, cnt_p = _sc_agg_counts(row, col, edge_weight, x_pad, zrow, zcnt)
  cnt_p3 = cnt_p.reshape(NC, N_PAD, 1)
  h1 = _tc_combine(acc_p, cnt_p3, x_pad, W1.T)

  acc_p2 = _sc_agg_plain(row, col, edge_weight, h1, zrow, zcnt)
  h2 = _tc_combine(acc_p2, cnt_p3, h1, W2.T)
  return h2[:N]


# speedup vs baseline: 4.1107x; 4.1107x over previous
import functools

import jax
import jax.numpy as jnp
from jax import lax
from jax.experimental import pallas as pl
from jax.experimental.pallas import tpu as pltpu
from jax.experimental.pallas import tpu_sc as plsc

N = 10000
E = 320000
D = 128
CELL_LEN = 100
SAME_W = 0.3
CROSS_W = 1.0

NC = 2
NS = 16
NW = NC * NS
EPW = E // NW
C = 80
NCHUNK = EPW // C
N_PAD = 10240
ZPT = N_PAD // NS
LANES = 16


@functools.lru_cache(maxsize=None)
def _sc_agg(with_counts):
  mesh = plsc.VectorSubcoreMesh(core_axis_name="c", subcore_axis_name="s")
  out_type = [jax.ShapeDtypeStruct((NC, N_PAD, D), jnp.float32)]
  if with_counts:
    out_type.append(jax.ShapeDtypeStruct((NC, N_PAD), jnp.float32))
  scratch = [
      pltpu.VMEM_SHARED((N_PAD, D), jnp.float32),
      pltpu.VMEM_SHARED((N_PAD,), jnp.float32),
      pltpu.VMEM((C,), jnp.int32),
      pltpu.VMEM((C,), jnp.int32),
      pltpu.VMEM((C,), jnp.float32),
      pltpu.VMEM((C,), jnp.float32),
      pltpu.VMEM((C,), jnp.float32),
      pltpu.VMEM((C, D), jnp.float32),
  ]

  def body(row_h, col_h, ew_h, x_h, zrow_h, zcnt_h, *rest):
    if with_counts:
      out_acc, out_cnt = rest[0], rest[1]
      scr = rest[2:]
    else:
      out_acc = rest[0]
      scr = rest[1:]
    acc_s, cnt_s, idx_r, idx_c, ew_v, w_v, ones_v, rows_v = scr

    cid = lax.axis_index("c")
    sid = lax.axis_index("s")
    wid = sid * NC + cid

    lo = sid * ZPT
    pltpu.sync_copy(zrow_h.at[pl.ds(lo, ZPT), :], acc_s.at[pl.ds(lo, ZPT), :])
    if with_counts:
      pltpu.sync_copy(zcnt_h.at[pl.ds(lo, ZPT)], cnt_s.at[pl.ds(lo, ZPT)])
    for j in range(C // LANES):
      ones_v[pl.ds(j * LANES, LANES)] = jnp.ones((LANES,), jnp.float32)
    plsc.subcore_barrier()

    def chunk_body(k, carry):
      base = pl.multiple_of(wid * EPW + k * C, 8)
      pltpu.sync_copy(row_h.at[pl.ds(base, C)], idx_r)
      pltpu.sync_copy(col_h.at[pl.ds(base, C)], idx_c)
      pltpu.sync_copy(ew_h.at[pl.ds(base, C)], ew_v)

      for j in range(C // LANES):
        sl = pl.ds(j * LANES, LANES)
        r16 = idx_r[sl]
        c16 = idx_c[sl]
        rt = jnp.where(r16 <= CELL_LEN, jnp.float32(1.0), jnp.float32(0.0))
        ct = jnp.where(c16 <= CELL_LEN, jnp.float32(1.0), jnp.float32(0.0))
        diff = jnp.abs(rt - ct)
        tw = jnp.float32(SAME_W) + jnp.float32(CROSS_W - SAME_W) * diff
        w_v[sl] = tw * ew_v[sl]

      pltpu.sync_copy(x_h.at[idx_r], rows_v)

      for j in range(C // LANES):
        w16 = w_v[pl.ds(j * LANES, LANES)]
        for l in range(LANES):
          r = j * LANES + l
          wr = w16[l]
          for d in range(D // LANES):
            sl = pl.ds(d * LANES, LANES)
            rows_v[r, sl] = rows_v[r, sl] * wr

      pltpu.sync_copy(rows_v, acc_s.at[idx_c], add=True)
      if with_counts:
        pltpu.sync_copy(ones_v, cnt_s.at[idx_c], add=True)
      return carry

    lax.fori_loop(0, NCHUNK, chunk_body, 0)
    plsc.subcore_barrier()

    pltpu.sync_copy(acc_s.at[pl.ds(lo, ZPT), :], out_acc.at[cid, pl.ds(lo, ZPT), :])
    if with_counts:
      pltpu.sync_copy(cnt_s.at[pl.ds(lo, ZPT)], out_cnt.at[cid, pl.ds(lo, ZPT)])

  return pl.kernel(body, out_type=out_type, mesh=mesh, scratch_types=scratch)


BN = 640


def _combine_body(acc_ref, cnt_ref, x_ref, wt_ref, o_ref):
  acc = acc_ref[0] + acc_ref[1]
  cnt = cnt_ref[0] + cnt_ref[1]
  inv = 1.0 / jnp.maximum(cnt, 1.0)
  o_ref[...] = acc * inv + jnp.dot(
      x_ref[...], wt_ref[...], preferred_element_type=jnp.float32)


def _tc_combine(acc_p, cnt_p, x_pad, wt):
  grid = (N_PAD // BN,)
  return pl.pallas_call(
      _combine_body,
      grid=grid,
      in_specs=[
          pl.BlockSpec((NC, BN, D), lambda i: (0, i, 0)),
          pl.BlockSpec((NC, BN, 1), lambda i: (0, i, 0)),
          pl.BlockSpec((BN, D), lambda i: (i, 0)),
          pl.BlockSpec((D, D), lambda i: (0, 0)),
      ],
      out_specs=pl.BlockSpec((BN, D), lambda i: (i, 0)),
      out_shape=jax.ShapeDtypeStruct((N_PAD, D), jnp.float32),
  )(acc_p, cnt_p, x_pad, wt)


def kernel(x, edge_index, edge_weight, W1, W2):
  row = edge_index[0]
  col = edge_index[1]
  x_pad = jnp.pad(x, ((0, N_PAD - N), (0, 0)))
  zrow = jnp.zeros((N_PAD, D), jnp.float32)
  zcnt = jnp.zeros((N_PAD,), jnp.float32)

  acc_p, cnt_p = _sc_agg(True)(row, col, edge_weight, x_pad, zrow, zcnt)
  cnt_p3 = cnt_p.reshape(NC, N_PAD, 1)
  h1 = _tc_combine(acc_p, cnt_p3, x_pad, W1.T)

  (acc_p2,) = _sc_agg(False)(row, col, edge_weight, h1, zrow, zcnt)
  h2 = _tc_combine(acc_p2, cnt_p3, h1, W2.T)
  return h2[:N]


# speedup vs baseline: 10.3989x; 2.5297x over previous
import functools

import jax
import jax.numpy as jnp
from jax import lax
from jax.experimental import pallas as pl
from jax.experimental.pallas import tpu as pltpu
from jax.experimental.pallas import tpu_sc as plsc

N = 10000
E = 320000
D = 128
CELL_LEN = 100
SAME_W = 0.3
CROSS_W = 1.0

NC = 2
NS = 16
NW = NC * NS
EPW = E // NW
C = 80
NCHUNK = EPW // C
N_PAD = 10240
ZPT = N_PAD // NS
LANES = 16
NBUF = 3


@functools.lru_cache(maxsize=None)
def _sc_agg(with_counts):
  mesh = plsc.VectorSubcoreMesh(core_axis_name="c", subcore_axis_name="s")
  out_type = [jax.ShapeDtypeStruct((NC, N_PAD, D), jnp.float32)]
  if with_counts:
    out_type.append(jax.ShapeDtypeStruct((NC, N_PAD), jnp.float32))
  scratch = [
      pltpu.VMEM_SHARED((N_PAD, D), jnp.float32),
      pltpu.VMEM_SHARED((N_PAD,), jnp.float32),
      pltpu.VMEM((NBUF * C, D), jnp.float32),
      pltpu.VMEM((3 * NBUF, C), jnp.int32),
      pltpu.VMEM((NBUF, C), jnp.int32),
      pltpu.VMEM((C,), jnp.float32),
  ] + [pltpu.SemaphoreType.DMA for _ in range(3 * NBUF + NBUF)]

  def body(epk_h, x_h, zrow_h, zcnt_h, *rest):
    if with_counts:
      out_acc, out_cnt = rest[0], rest[1]
      scr = rest[2:]
    else:
      out_acc = rest[0]
      scr = rest[1:]
    acc_s, cnt_s, rows_p, ebuf, cidx, ones_v = scr[:6]
    sems = scr[6:]
    se = sems[:NBUF]
    sg = sems[NBUF:2 * NBUF]
    ss = sems[2 * NBUF:3 * NBUF]
    sc = sems[3 * NBUF:4 * NBUF]

    cid = lax.axis_index("c")
    sid = lax.axis_index("s")
    wid = sid * NC + cid

    lo = sid * ZPT
    pltpu.sync_copy(zrow_h.at[pl.ds(lo, ZPT), :], acc_s.at[pl.ds(lo, ZPT), :])
    if with_counts:
      pltpu.sync_copy(zcnt_h.at[pl.ds(lo, ZPT)], cnt_s.at[pl.ds(lo, ZPT)])
    for j in range(C // LANES):
      ones_v[pl.ds(j * LANES, LANES)] = jnp.ones((LANES,), jnp.float32)
    plsc.subcore_barrier()

    def rbuf(b):
      return rows_p.at[pl.ds(b * C, C), :]

    def e_start(k, b):
      pltpu.async_copy(epk_h.at[wid, k], ebuf.at[pl.ds(3 * b, 3), :], se[b])

    def e_wait(k, b):
      pltpu.make_async_copy(
          epk_h.at[wid, k], ebuf.at[pl.ds(3 * b, 3), :], se[b]).wait()

    def g_start(k, b):
      pltpu.async_copy(x_h.at[ebuf.at[3 * b]], rbuf(b), sg[b])

    def g_wait(k, b):
      pltpu.make_async_copy(x_h.at[ebuf.at[3 * b]], rbuf(b), sg[b]).wait()

    def s_start(k, b):
      pltpu.async_copy(rbuf(b), acc_s.at[cidx.at[b]], ss[b], add=True)
      if with_counts:
        pltpu.async_copy(ones_v, cnt_s.at[cidx.at[b]], sc[b], add=True)

    def s_wait(k, b):
      pltpu.make_async_copy(rbuf(b), acc_s.at[cidx.at[b]], ss[b]).wait()
      if with_counts:
        pltpu.make_async_copy(ones_v, cnt_s.at[cidx.at[b]], sc[b]).wait()

    def scale(k, b):
      def jbody(j, carry):
        sl = pl.ds(j * LANES, LANES)
        r16 = ebuf[3 * b, sl]
        c16 = ebuf[3 * b + 1, sl]
        ew16 = lax.bitcast_convert_type(ebuf[3 * b + 2, sl], jnp.float32)
        cidx[b, sl] = c16
        rt = jnp.where(r16 <= CELL_LEN, jnp.float32(1.0), jnp.float32(0.0))
        ct = jnp.where(c16 <= CELL_LEN, jnp.float32(1.0), jnp.float32(0.0))
        diff = jnp.abs(rt - ct)
        w16 = (jnp.float32(SAME_W)
               + jnp.float32(CROSS_W - SAME_W) * diff) * ew16
        for l in range(LANES):
          wr = w16[l]
          r = b * C + j * LANES + l
          for d in range(D // LANES):
            s2 = pl.ds(d * LANES, LANES)
            rows_p[r, s2] = rows_p[r, s2] * wr
        return carry

      lax.fori_loop(0, C // LANES, jbody, 0)

    def step(k, b, wait_prev, load_next2):
      if wait_prev:
        s_wait(k - 2, (b + 1) % NBUF)
      if load_next2:
        e_start(k + 2, (b + 2) % NBUF)
      e_wait(k + 1, (b + 1) % NBUF)
      g_start(k + 1, (b + 1) % NBUF)
      g_wait(k, b)
      scale(k, b)
      s_start(k, b)

    e_start(0, 0)
    e_start(1, 1)
    e_wait(0, 0)
    g_start(0, 0)
    step(0, 0, False, True)
    step(1, 1, False, True)
    step(2, 2, True, True)

    def tri_body(p, carry):
      k = 3 * p
      step(k, 0, True, True)
      step(k + 1, 1, True, True)
      step(k + 2, 2, True, True)
      return carry

    lax.fori_loop(1, (NCHUNK - 2) // 3, tri_body, 0)

    s_wait(121, 1)
    e_wait(124, 1)
    g_start(124, 1)
    g_wait(123, 0)
    scale(123, 0)
    s_start(123, 0)
    s_wait(122, 2)
    g_wait(124, 1)
    scale(124, 1)
    s_start(124, 1)
    s_wait(123, 0)
    s_wait(124, 1)

    plsc.subcore_barrier()

    pltpu.sync_copy(acc_s.at[pl.ds(lo, ZPT), :], out_acc.at[cid, pl.ds(lo, ZPT), :])
    if with_counts:
      pltpu.sync_copy(cnt_s.at[pl.ds(lo, ZPT)], out_cnt.at[cid, pl.ds(lo, ZPT)])

  return pl.kernel(body, out_type=out_type, mesh=mesh, scratch_types=scratch)


BN = 640


def _mm_body(x_ref, wt_ref, o_ref):
  o_ref[...] = jnp.dot(x_ref[...], wt_ref[...],
                       preferred_element_type=jnp.float32)


def _tc_mm(x_pad, wt):
  return pl.pallas_call(
      _mm_body,
      grid=(N_PAD // BN,),
      in_specs=[
          pl.BlockSpec((BN, D), lambda i: (i, 0)),
          pl.BlockSpec((D, D), lambda i: (0, 0)),
      ],
      out_specs=pl.BlockSpec((BN, D), lambda i: (i, 0)),
      out_shape=jax.ShapeDtypeStruct((N_PAD, D), jnp.float32),
  )(x_pad, wt)


def _add_body(acc_ref, cnt_ref, m_ref, o_ref):
  acc = acc_ref[0] + acc_ref[1]
  cnt = cnt_ref[0] + cnt_ref[1]
  inv = 1.0 / jnp.maximum(cnt, 1.0)
  o_ref[...] = acc * inv + m_ref[...]


def _tc_add(acc_p, cnt_p, m):
  return pl.pallas_call(
      _add_body,
      grid=(N_PAD // BN,),
      in_specs=[
          pl.BlockSpec((NC, BN, D), lambda i: (0, i, 0)),
          pl.BlockSpec((NC, BN, 1), lambda i: (0, i, 0)),
          pl.BlockSpec((BN, D), lambda i: (i, 0)),
      ],
      out_specs=pl.BlockSpec((BN, D), lambda i: (i, 0)),
      out_shape=jax.ShapeDtypeStruct((N_PAD, D), jnp.float32),
  )(acc_p, cnt_p, m)


def kernel(x, edge_index, edge_weight, W1, W2):
  rowr = edge_index[0].reshape(NW, NCHUNK, C)
  colr = edge_index[1].reshape(NW, NCHUNK, C)
  ewr = lax.bitcast_convert_type(edge_weight, jnp.int32).reshape(NW, NCHUNK, C)
  epk = jnp.stack([rowr, colr, ewr], axis=2)
  x_pad = jnp.pad(x, ((0, N_PAD - N), (0, 0)))
  zrow = jnp.zeros((N_PAD, D), jnp.float32)
  zcnt = jnp.zeros((N_PAD,), jnp.float32)

  m1 = _tc_mm(x_pad, W1.T)
  acc_p, cnt_p = _sc_agg(True)(epk, x_pad, zrow, zcnt)
  cnt_p3 = cnt_p.reshape(NC, N_PAD, 1)
  h1 = _tc_add(acc_p, cnt_p3, m1)

  m2 = _tc_mm(h1, W2.T)
  (acc_p2,) = _sc_agg(False)(epk, h1, zrow, zcnt)
  h2 = _tc_add(acc_p2, cnt_p3, m2)
  return h2[:N]


# speedup vs baseline: 12.3051x; 1.1833x over previous
import functools

import jax
import jax.numpy as jnp
from jax import lax
from jax.experimental import pallas as pl
from jax.experimental.pallas import tpu as pltpu
from jax.experimental.pallas import tpu_sc as plsc

N = 10000
E = 320000
D = 128
CELL_LEN = 100
SAME_W = 0.3
CROSS_W = 1.0

NC = 2
NS = 16
NW = NC * NS
EPW = E // NW
C = 80
NCHUNK = EPW // C
N_PAD = 10240
ZPT = N_PAD // NS
LANES = 16
NBUF = 3


@functools.lru_cache(maxsize=None)
def _sc_agg(with_counts):
  mesh = plsc.VectorSubcoreMesh(core_axis_name="c", subcore_axis_name="s")
  out_type = [jax.ShapeDtypeStruct((NC, N_PAD, D), jnp.float32)]
  if with_counts:
    out_type.append(jax.ShapeDtypeStruct((NC, N_PAD), jnp.float32))
  scratch = [
      pltpu.VMEM_SHARED((N_PAD, D), jnp.float32),
      pltpu.VMEM_SHARED((N_PAD,), jnp.float32),
      pltpu.VMEM((NBUF * C, D), jnp.float32),
      pltpu.VMEM((NBUF, C), jnp.int32),
      pltpu.VMEM((NBUF, C), jnp.int32),
      pltpu.VMEM((NBUF, C), jnp.float32),
      pltpu.VMEM((NBUF, C), jnp.int32),
      pltpu.VMEM((ZPT,), jnp.float32),
  ] + [pltpu.SemaphoreType.DMA for _ in range(5 * NBUF)]

  def body(row_h, col_h, ew_h, x_h, *rest):
    if with_counts:
      out_acc, out_cnt = rest[0], rest[1]
      scr = rest[2:]
    else:
      out_acc = rest[0]
      scr = rest[1:]
    acc_s, cnt_s, rows_p, rib, cib, ewb, cidx, zo_v = scr[:8]
    sems = scr[8:]
    se = sems[:NBUF]
    sw = sems[NBUF:2 * NBUF]
    sg = sems[2 * NBUF:3 * NBUF]
    ss = sems[3 * NBUF:4 * NBUF]
    sc = sems[4 * NBUF:5 * NBUF]

    cid = lax.axis_index("c")
    sid = lax.axis_index("s")
    wid = sid * NC + cid
    ebase = wid * EPW

    def zrow(i, carry):
      for d in range(D // LANES):
        rows_p[i, pl.ds(d * LANES, LANES)] = jnp.zeros((LANES,), jnp.float32)
      return carry

    lax.fori_loop(0, NBUF * C, zrow, 0)
    for j in range(ZPT // LANES):
      zo_v[pl.ds(j * LANES, LANES)] = jnp.zeros((LANES,), jnp.float32)

    lo = sid * ZPT
    nz = NBUF * C
    pltpu.sync_copy(rows_p, acc_s.at[pl.ds(lo, nz), :])
    pltpu.sync_copy(rows_p, acc_s.at[pl.ds(lo + nz, nz), :])
    pltpu.sync_copy(rows_p.at[pl.ds(0, ZPT - 2 * nz), :],
                    acc_s.at[pl.ds(lo + 2 * nz, ZPT - 2 * nz), :])
    if with_counts:
      pltpu.sync_copy(zo_v, cnt_s.at[pl.ds(lo, ZPT)])
      for j in range(C // LANES):
        zo_v[pl.ds(j * LANES, LANES)] = jnp.ones((LANES,), jnp.float32)
    plsc.subcore_barrier()

    def rbuf(b):
      return rows_p.at[pl.ds(b * C, C), :]

    def ones_v():
      return zo_v.at[pl.ds(0, C)]

    def e_start(k, b):
      base = pl.multiple_of(ebase + k * C, 8)
      pltpu.async_copy(row_h.at[pl.ds(base, C)], rib.at[b], se[b])
      pltpu.async_copy(col_h.at[pl.ds(base, C)], cib.at[b], se[b])
      pltpu.async_copy(ew_h.at[pl.ds(base, C)], ewb.at[b], sw[b])

    def e_wait(k, b):
      base = pl.multiple_of(ebase + k * C, 8)
      pltpu.make_async_copy(row_h.at[pl.ds(base, C)], rib.at[b], se[b]).wait()
      pltpu.make_async_copy(col_h.at[pl.ds(base, C)], cib.at[b], se[b]).wait()
      pltpu.make_async_copy(ew_h.at[pl.ds(base, C)], ewb.at[b], sw[b]).wait()

    def g_start(k, b):
      pltpu.async_copy(x_h.at[rib.at[b]], rbuf(b), sg[b])

    def g_wait(k, b):
      pltpu.make_async_copy(x_h.at[rib.at[b]], rbuf(b), sg[b]).wait()

    def s_start(k, b):
      pltpu.async_copy(rbuf(b), acc_s.at[cidx.at[b]], ss[b], add=True)
      if with_counts:
        pltpu.async_copy(ones_v(), cnt_s.at[cidx.at[b]], sc[b], add=True)

    def s_wait(k, b):
      pltpu.make_async_copy(rbuf(b), acc_s.at[cidx.at[b]], ss[b]).wait()
      if with_counts:
        pltpu.make_async_copy(ones_v(), cnt_s.at[cidx.at[b]], sc[b]).wait()

    def scale(k, b):
      def jbody(j, carry):
        sl = pl.ds(j * LANES, LANES)
        r16 = rib[b, sl]
        c16 = cib[b, sl]
        ew16 = ewb[b, sl]
        cidx[b, sl] = c16
        rt = jnp.where(r16 <= CELL_LEN, jnp.float32(1.0), jnp.float32(0.0))
        ct = jnp.where(c16 <= CELL_LEN, jnp.float32(1.0), jnp.float32(0.0))
        diff = jnp.abs(rt - ct)
        w16 = (jnp.float32(SAME_W)
               + jnp.float32(CROSS_W - SAME_W) * diff) * ew16
        for l in range(LANES):
          wr = w16[l]
          r = b * C + j * LANES + l
          for d in range(D // LANES):
            s2 = pl.ds(d * LANES, LANES)
            rows_p[r, s2] = rows_p[r, s2] * wr
        return carry

      lax.fori_loop(0, C // LANES, jbody, 0)

    def step(k, b, wait_prev, load_next2):
      if wait_prev:
        s_wait(k - 2, (b + 1) % NBUF)
      if load_next2:
        e_start(k + 2, (b + 2) % NBUF)
      e_wait(k + 1, (b + 1) % NBUF)
      g_start(k + 1, (b + 1) % NBUF)
      g_wait(k, b)
      scale(k, b)
      s_start(k, b)

    e_start(0, 0)
    e_start(1, 1)
    e_wait(0, 0)
    g_start(0, 0)
    step(0, 0, False, True)
    step(1, 1, False, True)
    step(2, 2, True, True)

    def tri_body(p, carry):
      k = 3 * p
      step(k, 0, True, True)
      step(k + 1, 1, True, True)
      step(k + 2, 2, True, True)
      return carry

    lax.fori_loop(1, (NCHUNK - 2) // 3, tri_body, 0)

    s_wait(121, 1)
    e_wait(124, 1)
    g_start(124, 1)
    g_wait(123, 0)
    scale(123, 0)
    s_start(123, 0)
    s_wait(122, 2)
    g_wait(124, 1)
    scale(124, 1)
    s_start(124, 1)
    s_wait(123, 0)
    s_wait(124, 1)

    plsc.subcore_barrier()

    pltpu.sync_copy(acc_s.at[pl.ds(lo, ZPT), :], out_acc.at[cid, pl.ds(lo, ZPT), :])
    if with_counts:
      pltpu.sync_copy(cnt_s.at[pl.ds(lo, ZPT)], out_cnt.at[cid, pl.ds(lo, ZPT)])

  return pl.kernel(body, out_type=out_type, mesh=mesh, scratch_types=scratch)


BN = 2000


def _mm_body(x_ref, w_ref, o_ref):
  o_ref[...] = lax.dot_general(
      x_ref[...], w_ref[...], (((1,), (1,)), ((), ())),
      preferred_element_type=jnp.float32)


def _tc_mm(x, w):
  return pl.pallas_call(
      _mm_body,
      grid=(N // BN,),
      in_specs=[
          pl.BlockSpec((BN, D), lambda i: (i, 0)),
          pl.BlockSpec((D, D), lambda i: (0, 0)),
      ],
      out_specs=pl.BlockSpec((BN, D), lambda i: (i, 0)),
      out_shape=jax.ShapeDtypeStruct((N, D), jnp.float32),
  )(x, w)


def _add_body(acc_ref, cnt_ref, m_ref, o_ref):
  acc = acc_ref[0] + acc_ref[1]
  cnt = cnt_ref[0] + cnt_ref[1]
  inv = 1.0 / jnp.maximum(cnt, 1.0)
  o_ref[...] = acc * inv + m_ref[...]


def _tc_add(acc_p, cnt_p, m):
  return pl.pallas_call(
      _add_body,
      grid=(N // BN,),
      in_specs=[
          pl.BlockSpec((NC, BN, D), lambda i: (0, i, 0)),
          pl.BlockSpec((NC, BN, 1), lambda i: (0, i, 0)),
          pl.BlockSpec((BN, D), lambda i: (i, 0)),
      ],
      out_specs=pl.BlockSpec((BN, D), lambda i: (i, 0)),
      out_shape=jax.ShapeDtypeStruct((N, D), jnp.float32),
  )(acc_p, cnt_p, m)


def kernel(x, edge_index, edge_weight, W1, W2):
  row = edge_index[0]
  col = edge_index[1]
  m1 = _tc_mm(x, W1)
  acc_p, cnt_p = _sc_agg(True)(row, col, edge_weight, x)
  cnt_p3 = cnt_p.reshape(NC, N_PAD, 1)
  h1 = _tc_add(acc_p, cnt_p3, m1)

  m2 = _tc_mm(h1, W2)
  (acc_p2,) = _sc_agg(False)(row, col, edge_weight, h1)
  h2 = _tc_add(acc_p2, cnt_p3, m2)
  return h2
